# baseline (speedup 1.0000x reference)
import jax
import jax.numpy as jnp
import numpy as np
from jax import lax
from jax.experimental import pallas as pl
from jax.experimental.pallas import tpu as pltpu
from jax.experimental.pallas import tpu_sc as plsc

_NUM_PATCHES = 1024
_N_PATHS = 2
_NPP = 512
_BATCH = 64
_C = 192

_NC = 2
_NS = 16
_NW = _NC * _NS

_CB = _C // 8
_STRIPS_PER_W = _BATCH * _CB // _NW
_L = 16
_G = _NPP // _L

_FLAT_PATCH_IDX = np.array([469, 751, 513, 685, 281, 151, 267, 881, 174, 329, 414, 525, 401, 321, 295, 446, 411, 360, 1016, 137, 548, 245, 865, 966, 171, 273, 440, 476, 572, 206, 701, 125, 617, 724, 436, 929, 494, 40, 723, 493, 595, 398, 524, 683, 28, 877, 647, 728, 104, 441, 547, 58, 319, 483, 822, 950, 597, 482, 94, 61, 861, 423, 165, 294, 318, 552, 277, 131, 740, 408, 598, 843, 732, 532, 206, 514, 447, 64, 608, 287, 728, 737, 693, 78, 610, 609, 103, 102, 59, 124, 443, 471, 490, 642, 416, 998, 687, 184, 351, 323, 283, 143, 624, 103, 217, 798, 535, 556, 3, 860, 866, 388, 968, 613, 929, 263, 584, 140, 302, 197, 749, 383, 305, 485, 665, 1019, 218, 893, 229, 114, 67, 842, 69, 548, 540, 764, 969, 969, 695, 341, 232, 991, 65, 111, 629, 207, 860, 990, 792, 594, 759, 865, 958, 179, 12, 119, 917, 109, 710, 617, 953, 777, 746, 432, 755, 519, 885, 851, 266, 96, 188, 513, 577, 631, 415, 586, 95, 595, 810, 208, 508, 608, 706, 422, 456, 229, 518, 144, 38, 665, 87, 975, 657, 759, 82, 791, 976, 788, 279, 192, 903, 395, 765, 13, 659, 236, 805, 494, 789, 633, 568, 542, 887, 503, 891, 610, 264, 812, 273, 293, 893, 978, 137, 478, 806, 116, 529, 657, 646, 442, 122, 678, 775, 859, 193, 855, 868, 562, 737, 1018, 884, 348, 894, 302, 1010, 825, 474, 641, 801, 549, 400, 1000, 667, 89, 799, 220, 304, 299, 292, 472, 709, 387, 396, 815, 650, 882, 826, 1010, 766, 557, 690, 27, 683, 638, 4, 35, 689, 534, 488, 922, 398, 399, 107, 440, 238, 1004, 9, 660, 778, 1003, 359, 827, 310, 409, 159, 264, 478, 320, 760, 444, 462, 1011, 254, 630, 681, 23, 774, 363, 593, 266, 874, 895, 146, 589, 588, 715, 754, 330, 286, 871, 480, 829, 613, 672, 942, 317, 2, 129, 164, 4, 570, 196, 463, 652, 753, 191, 230, 150, 108, 862, 275, 960, 811, 853, 177, 591, 236, 138, 616, 426, 940, 624, 662, 177, 353, 529, 54, 328, 950, 164, 371, 903, 73, 84, 965, 410, 81, 168, 905, 569, 663, 130, 961, 904, 40, 826, 620, 940, 428, 644, 1022, 1014, 228, 785, 5, 337, 1005, 686, 410, 643, 423, 217, 869, 424, 362, 668, 558, 451, 45, 956, 704, 1017, 918, 880, 285, 280, 309, 160, 280, 883, 330, 563, 729, 694, 18, 492, 523, 858, 504, 876, 437, 526, 736, 582, 688, 891, 730, 970, 843, 840, 205, 135, 718, 487, 521, 5, 849, 24, 639, 693, 450, 745, 336, 539, 840, 405, 575, 136, 342, 93, 106, 272, 984, 839, 492, 670, 633, 972, 71, 288, 914, 296, 255, 646, 167, 161, 507, 285, 110, 915, 1013, 632, 405, 367, 417, 51, 1014, 800, 372, 45, 919, 474, 696, 844, 70, 607, 101, 1022, 707, 508, 745, 780, 26, 68, 14, 214, 519, 9, 821, 427, 84, 267, 435, 258, 123, 559, 882, 216, 421, 94, 888, 307, 370, 664, 454, 353, 722, 364, 322, 450, 111, 900, 311, 54, 780, 620, 549, 12, 253, 22, 752, 783, 397, 890, 604, 175, 395, 705, 248, 703, 543, 976, 92, 106, 282, 627, 39, 629, 626, 504, 375, 965, 880, 713, 276, 543, 458, 945, 848, 97, 100, 313, 345, 375, 927, 786, 224, 48, 812, 818, 226, 793, 183, 376, 459, 176, 35, 647, 757, 902, 901, 332, 625, 977, 98, 600, 594, 312, 857, 199, 387, 71, 682, 326, 324, 958, 247, 404, 390, 776, 388, 658, 52, 867, 424, 459, 795, 67, 419, 300, 294, 355, 908, 458, 461, 66, 612, 1020, 668, 246, 776, 244, 120, 531, 76, 794, 764, 789, 404, 808, 160, 460, 931, 157, 442, 65, 29, 869, 944, 1009, 468, 169, 772, 147, 714, 449, 751, 565, 420, 784, 939, 575, 551, 704, 648, 535, 72, 577, 936, 343, 191, 805, 727, 242, 830, 230, 406, 431, 870, 182, 995, 628, 252, 339, 298, 153, 352, 681, 946, 252, 113, 884, 623, 973, 198, 91, 586, 498, 808, 986, 838, 25, 389, 342, 636, 989, 33, 389, 833, 698, 839, 897, 583, 731, 858, 592, 853, 311, 855, 571, 534, 912, 213, 974, 879, 248, 308, 790, 603, 484, 158, 292, 153, 112, 496, 455, 482, 243, 272, 666, 118, 887, 555, 947, 53, 233, 559, 507, 8, 490, 495, 720, 325, 203, 713, 79, 520, 690, 642, 782, 530, 36, 470, 952, 97, 714, 214, 933, 427, 580, 11, 1002, 831, 417, 872, 579, 349, 21, 79, 210, 1001, 615, 877, 362, 169, 517, 418, 232, 133, 390, 219, 588, 566, 909, 105, 238, 803, 443, 173, 56, 109, 926, 1015, 730, 315, 928, 791, 108, 578, 301, 344, 118, 23, 999, 587, 701, 357, 953, 466, 438, 878, 469, 444, 204, 841, 421, 787, 295, 788, 10, 373, 828, 135, 171, 988, 692, 876, 816, 246, 799, 889, 604, 88, 370, 859, 654, 368, 454, 844, 69, 896, 846, 208, 801, 211, 634, 814, 707, 78, 921, 460, 946, 31, 371, 734, 166, 692, 530, 212, 319, 582, 939, 818, 852, 658, 163, 1, 183, 197, 344, 684, 963, 738, 854, 216, 467, 602, 803, 57, 688, 671, 250, 813, 425, 926, 406, 185, 522, 785, 31, 215, 152, 143, 566, 983, 568, 194, 76, 600, 73, 7, 995, 502, 30, 1003, 456, 916, 721, 816, 123, 86, 158, 355, 523, 807, 663, 743, 8, 131, 209, 234, 797, 75, 39, 994, 453, 385, 545, 711, 612, 666, 274, 895, 479, 479, 618, 121, 659, 377, 419, 56, 373, 265, 994, 312, 139, 957, 457, 697, 418, 1021, 497, 638, 971, 510, 188, 392, 462, 126, 41, 1017, 773, 673, 133, 303, 289, 425, 59, 299, 391, 998, 226, 705, 369, 134, 931, 567, 1, 828, 159, 429, 927, 771, 325, 403, 412, 515, 807, 676, 528, 341, 546, 13, 898, 125, 117, 851, 361, 321, 15, 471, 172, 220, 366, 240, 1021, 565, 413, 562, 753, 1020, 547, 259, 17, 132, 437, 991, 439, 249, 368, 794, 468, 413, 357], dtype=np.int32)

_COL_IDX = _FLAT_PATCH_IDX.reshape(_NPP, _N_PATHS).T.reshape(-1).copy()


def _lane_gather_body(x_hbm, idx_hbm, out_hbm, idx_v, in_v, out_v,
                      in_sems, out_sems):
    wid = lax.axis_index("s") * _NC + lax.axis_index("c")

    pltpu.sync_copy(idx_hbm, idx_v)

    def in_row(k):
        return k * 8

    def fetch(slot, k):
        return pltpu.async_copy(x_hbm.at[pl.ds(in_row(k), 8), :],
                                in_v.at[slot], in_sems.at[slot])

    def flush(slot, k):
        b = k // _CB
        cb = k - b * _CB
        handles = []
        for n in range(_N_PATHS):
            ko = (b * _N_PATHS + n) * _CB + cb
            handles.append(
                pltpu.async_copy(out_v.at[slot, n],
                                 out_hbm.at[pl.ds(ko * 8, 8), :],
                                 out_sems.at[slot, n]))
        return handles

    def compute(slot):
        @plsc.parallel_loop(0, _N_PATHS * _G, unroll=16)
        def _(gi):
            colv = idx_v[pl.ds(gi * _L, _L)]
            n = lax.shift_right_logical(gi, 5)
            joff = lax.bitwise_and(gi, 31) * _L
            for c in range(8):
                rowv = jnp.full((_L,), c, jnp.int32)
                val = plsc.load_gather(in_v.at[slot], [rowv, colv])
                out_v[slot, n, c, pl.ds(joff, _L)] = val

    base = wid * _STRIPS_PER_W

    fetch(0, base + 0)
    fetch(1, base + 1)

    def strip_step(i, carry):
        slot = lax.rem(i, 2)
        k = base + i
        pltpu.make_async_copy(x_hbm.at[pl.ds(in_row(k), 8), :],
                              in_v.at[slot], in_sems.at[slot]).wait()
        @pl.when(i >= 2)
        def _():
            for n in range(_N_PATHS):
                pltpu.make_async_copy(out_v.at[slot, n],
                                      out_hbm.at[pl.ds(0, 8), :],
                                      out_sems.at[slot, n]).wait()
        compute(slot)
        flush(slot, k)
        @pl.when(i + 2 < _STRIPS_PER_W)
        def _():
            fetch(slot, k + 2)
        return carry

    lax.fori_loop(0, _STRIPS_PER_W, strip_step, 0)

    for i in range(_STRIPS_PER_W - 2, _STRIPS_PER_W):
        slot = i % 2
        for n in range(_N_PATHS):
            pltpu.make_async_copy(out_v.at[slot, n],
                                  out_hbm.at[pl.ds(0, 8), :],
                                  out_sems.at[slot, n]).wait()


_lane_gather = pl.kernel(
    _lane_gather_body,
    out_type=jax.ShapeDtypeStruct((_BATCH * _N_PATHS * _C, _NPP), jnp.float32),
    mesh=plsc.VectorSubcoreMesh(core_axis_name="c", subcore_axis_name="s"),
    compiler_params=pltpu.CompilerParams(use_tc_tiling_on_sc=True,
                                         needs_layout_passes=False),
    scratch_types=[
        pltpu.VMEM((_N_PATHS * _NPP,), jnp.int32),
        pltpu.VMEM((2, 8, _NUM_PATCHES), jnp.float32),
        pltpu.VMEM((2, _N_PATHS, 8, _NPP), jnp.float32),
        pltpu.SemaphoreType.DMA((2,)),
        pltpu.SemaphoreType.DMA((2, _N_PATHS)),
    ],
)


def kernel(inputs):
    x_t = jnp.transpose(inputs, (0, 2, 1)).reshape(_BATCH * _C, _NUM_PATCHES)
    col_idx = jnp.asarray(_COL_IDX)
    out_t = _lane_gather(x_t, col_idx)
    out_t = out_t.reshape(_BATCH, _N_PATHS, _C, _NPP)
    return jnp.transpose(out_t, (0, 3, 1, 2))


# speedup vs baseline: 1.0470x; 1.0470x over previous
import jax
import jax.numpy as jnp
import numpy as np
from jax import lax
from jax.experimental import pallas as pl
from jax.experimental.pallas import tpu as pltpu
from jax.experimental.pallas import tpu_sc as plsc

_NUM_PATCHES = 1024
_N_PATHS = 2
_NPP = 512
_BATCH = 64
_C = 192

_NC = 2
_NS = 16
_NW = _NC * _NS

_CB = _C // 8
_STRIPS_PER_W = _BATCH * _CB // _NW
_L = 16
_G = _NPP // _L

_FLAT_PATCH_IDX = np.array([469, 751, 513, 685, 281, 151, 267, 881, 174, 329, 414, 525, 401, 321, 295, 446, 411, 360, 1016, 137, 548, 245, 865, 966, 171, 273, 440, 476, 572, 206, 701, 125, 617, 724, 436, 929, 494, 40, 723, 493, 595, 398, 524, 683, 28, 877, 647, 728, 104, 441, 547, 58, 319, 483, 822, 950, 597, 482, 94, 61, 861, 423, 165, 294, 318, 552, 277, 131, 740, 408, 598, 843, 732, 532, 206, 514, 447, 64, 608, 287, 728, 737, 693, 78, 610, 609, 103, 102, 59, 124, 443, 471, 490, 642, 416, 998, 687, 184, 351, 323, 283, 143, 624, 103, 217, 798, 535, 556, 3, 860, 866, 388, 968, 613, 929, 263, 584, 140, 302, 197, 749, 383, 305, 485, 665, 1019, 218, 893, 229, 114, 67, 842, 69, 548, 540, 764, 969, 969, 695, 341, 232, 991, 65, 111, 629, 207, 860, 990, 792, 594, 759, 865, 958, 179, 12, 119, 917, 109, 710, 617, 953, 777, 746, 432, 755, 519, 885, 851, 266, 96, 188, 513, 577, 631, 415, 586, 95, 595, 810, 208, 508, 608, 706, 422, 456, 229, 518, 144, 38, 665, 87, 975, 657, 759, 82, 791, 976, 788, 279, 192, 903, 395, 765, 13, 659, 236, 805, 494, 789, 633, 568, 542, 887, 503, 891, 610, 264, 812, 273, 293, 893, 978, 137, 478, 806, 116, 529, 657, 646, 442, 122, 678, 775, 859, 193, 855, 868, 562, 737, 1018, 884, 348, 894, 302, 1010, 825, 474, 641, 801, 549, 400, 1000, 667, 89, 799, 220, 304, 299, 292, 472, 709, 387, 396, 815, 650, 882, 826, 1010, 766, 557, 690, 27, 683, 638, 4, 35, 689, 534, 488, 922, 398, 399, 107, 440, 238, 1004, 9, 660, 778, 1003, 359, 827, 310, 409, 159, 264, 478, 320, 760, 444, 462, 1011, 254, 630, 681, 23, 774, 363, 593, 266, 874, 895, 146, 589, 588, 715, 754, 330, 286, 871, 480, 829, 613, 672, 942, 317, 2, 129, 164, 4, 570, 196, 463, 652, 753, 191, 230, 150, 108, 862, 275, 960, 811, 853, 177, 591, 236, 138, 616, 426, 940, 624, 662, 177, 353, 529, 54, 328, 950, 164, 371, 903, 73, 84, 965, 410, 81, 168, 905, 569, 663, 130, 961, 904, 40, 826, 620, 940, 428, 644, 1022, 1014, 228, 785, 5, 337, 1005, 686, 410, 643, 423, 217, 869, 424, 362, 668, 558, 451, 45, 956, 704, 1017, 918, 880, 285, 280, 309, 160, 280, 883, 330, 563, 729, 694, 18, 492, 523, 858, 504, 876, 437, 526, 736, 582, 688, 891, 730, 970, 843, 840, 205, 135, 718, 487, 521, 5, 849, 24, 639, 693, 450, 745, 336, 539, 840, 405, 575, 136, 342, 93, 106, 272, 984, 839, 492, 670, 633, 972, 71, 288, 914, 296, 255, 646, 167, 161, 507, 285, 110, 915, 1013, 632, 405, 367, 417, 51, 1014, 800, 372, 45, 919, 474, 696, 844, 70, 607, 101, 1022, 707, 508, 745, 780, 26, 68, 14, 214, 519, 9, 821, 427, 84, 267, 435, 258, 123, 559, 882, 216, 421, 94, 888, 307, 370, 664, 454, 353, 722, 364, 322, 450, 111, 900, 311, 54, 780, 620, 549, 12, 253, 22, 752, 783, 397, 890, 604, 175, 395, 705, 248, 703, 543, 976, 92, 106, 282, 627, 39, 629, 626, 504, 375, 965, 880, 713, 276, 543, 458, 945, 848, 97, 100, 313, 345, 375, 927, 786, 224, 48, 812, 818, 226, 793, 183, 376, 459, 176, 35, 647, 757, 902, 901, 332, 625, 977, 98, 600, 594, 312, 857, 199, 387, 71, 682, 326, 324, 958, 247, 404, 390, 776, 388, 658, 52, 867, 424, 459, 795, 67, 419, 300, 294, 355, 908, 458, 461, 66, 612, 1020, 668, 246, 776, 244, 120, 531, 76, 794, 764, 789, 404, 808, 160, 460, 931, 157, 442, 65, 29, 869, 944, 1009, 468, 169, 772, 147, 714, 449, 751, 565, 420, 784, 939, 575, 551, 704, 648, 535, 72, 577, 936, 343, 191, 805, 727, 242, 830, 230, 406, 431, 870, 182, 995, 628, 252, 339, 298, 153, 352, 681, 946, 252, 113, 884, 623, 973, 198, 91, 586, 498, 808, 986, 838, 25, 389, 342, 636, 989, 33, 389, 833, 698, 839, 897, 583, 731, 858, 592, 853, 311, 855, 571, 534, 912, 213, 974, 879, 248, 308, 790, 603, 484, 158, 292, 153, 112, 496, 455, 482, 243, 272, 666, 118, 887, 555, 947, 53, 233, 559, 507, 8, 490, 495, 720, 325, 203, 713, 79, 520, 690, 642, 782, 530, 36, 470, 952, 97, 714, 214, 933, 427, 580, 11, 1002, 831, 417, 872, 579, 349, 21, 79, 210, 1001, 615, 877, 362, 169, 517, 418, 232, 133, 390, 219, 588, 566, 909, 105, 238, 803, 443, 173, 56, 109, 926, 1015, 730, 315, 928, 791, 108, 578, 301, 344, 118, 23, 999, 587, 701, 357, 953, 466, 438, 878, 469, 444, 204, 841, 421, 787, 295, 788, 10, 373, 828, 135, 171, 988, 692, 876, 816, 246, 799, 889, 604, 88, 370, 859, 654, 368, 454, 844, 69, 896, 846, 208, 801, 211, 634, 814, 707, 78, 921, 460, 946, 31, 371, 734, 166, 692, 530, 212, 319, 582, 939, 818, 852, 658, 163, 1, 183, 197, 344, 684, 963, 738, 854, 216, 467, 602, 803, 57, 688, 671, 250, 813, 425, 926, 406, 185, 522, 785, 31, 215, 152, 143, 566, 983, 568, 194, 76, 600, 73, 7, 995, 502, 30, 1003, 456, 916, 721, 816, 123, 86, 158, 355, 523, 807, 663, 743, 8, 131, 209, 234, 797, 75, 39, 994, 453, 385, 545, 711, 612, 666, 274, 895, 479, 479, 618, 121, 659, 377, 419, 56, 373, 265, 994, 312, 139, 957, 457, 697, 418, 1021, 497, 638, 971, 510, 188, 392, 462, 126, 41, 1017, 773, 673, 133, 303, 289, 425, 59, 299, 391, 998, 226, 705, 369, 134, 931, 567, 1, 828, 159, 429, 927, 771, 325, 403, 412, 515, 807, 676, 528, 341, 546, 13, 898, 125, 117, 851, 361, 321, 15, 471, 172, 220, 366, 240, 1021, 565, 413, 562, 753, 1020, 547, 259, 17, 132, 437, 991, 439, 249, 368, 794, 468, 413, 357], dtype=np.int32)

_COL_IDX = _FLAT_PATCH_IDX.reshape(_NPP, _N_PATHS).T.reshape(-1).copy()


def _lane_gather_body(x_hbm, idx_hbm, out_hbm, idx_v, in_v, out_v,
                      in_sems, out_sems):
    wid = lax.axis_index("s") * _NC + lax.axis_index("c")

    pltpu.sync_copy(idx_hbm, idx_v)

    def in_row(k):
        return k * 8

    def fetch(slot, k):
        return pltpu.async_copy(x_hbm.at[pl.ds(in_row(k), 8), :],
                                in_v.at[slot], in_sems.at[slot])

    def flush(slot, k):
        b = k // _CB
        cb = k - b * _CB
        handles = []
        for n in range(_N_PATHS):
            ko = (b * _N_PATHS + n) * _CB + cb
            handles.append(
                pltpu.async_copy(out_v.at[slot, n],
                                 out_hbm.at[pl.ds(ko * 8, 8), :],
                                 out_sems.at[slot, n]))
        return handles

    def compute(slot):
        @plsc.parallel_loop(0, _N_PATHS * _G, unroll=8)
        def _(gi):
            colv = idx_v[pl.ds(gi * _L, _L)]
            n = lax.shift_right_logical(gi, 5)
            joff = lax.bitwise_and(gi, 31) * _L
            for c in range(8):
                rowv = jnp.full((_L,), c, jnp.int32)
                val = plsc.load_gather(in_v.at[slot], [rowv, colv])
                out_v[slot, n, c, pl.ds(joff, _L)] = val

    base = wid * _STRIPS_PER_W

    fetch(0, base + 0)
    fetch(1, base + 1)

    def strip_step(i, carry):
        slot = lax.rem(i, 2)
        k = base + i
        pltpu.make_async_copy(x_hbm.at[pl.ds(in_row(k), 8), :],
                              in_v.at[slot], in_sems.at[slot]).wait()
        @pl.when(i >= 2)
        def _():
            for n in range(_N_PATHS):
                pltpu.make_async_copy(out_v.at[slot, n],
                                      out_hbm.at[pl.ds(0, 8), :],
                                      out_sems.at[slot, n]).wait()
        compute(slot)
        flush(slot, k)
        @pl.when(i + 2 < _STRIPS_PER_W)
        def _():
            fetch(slot, k + 2)
        return carry

    lax.fori_loop(0, _STRIPS_PER_W, strip_step, 0)

    for i in range(_STRIPS_PER_W - 2, _STRIPS_PER_W):
        slot = i % 2
        for n in range(_N_PATHS):
            pltpu.make_async_copy(out_v.at[slot, n],
                                  out_hbm.at[pl.ds(0, 8), :],
                                  out_sems.at[slot, n]).wait()


_lane_gather = pl.kernel(
    _lane_gather_body,
    out_type=jax.ShapeDtypeStruct((_BATCH * _N_PATHS * _C, _NPP), jnp.float32),
    mesh=plsc.VectorSubcoreMesh(core_axis_name="c", subcore_axis_name="s"),
    compiler_params=pltpu.CompilerParams(use_tc_tiling_on_sc=True,
                                         needs_layout_passes=False),
    scratch_types=[
        pltpu.VMEM((_N_PATHS * _NPP,), jnp.int32),
        pltpu.VMEM((2, 8, _NUM_PATCHES), jnp.float32),
        pltpu.VMEM((2, _N_PATHS, 8, _NPP), jnp.float32),
        pltpu.SemaphoreType.DMA((2,)),
        pltpu.SemaphoreType.DMA((2, _N_PATHS)),
    ],
)


def kernel(inputs):
    x_t = jnp.transpose(inputs, (0, 2, 1)).reshape(_BATCH * _C, _NUM_PATCHES)
    col_idx = jnp.asarray(_COL_IDX)
    out_t = _lane_gather(x_t, col_idx)
    out_t = out_t.reshape(_BATCH, _N_PATHS, _C, _NPP)
    return jnp.transpose(out_t, (0, 3, 1, 2))


# speedup vs baseline: 1.1607x; 1.1086x over previous
import jax
import jax.numpy as jnp
import numpy as np
from jax import lax
from jax.experimental import pallas as pl
from jax.experimental.pallas import tpu as pltpu
from jax.experimental.pallas import tpu_sc as plsc

_NUM_PATCHES = 1024
_N_PATHS = 2
_NPP = 512
_BATCH = 64
_C = 192

_NC = 2
_NS = 16
_NW = _NC * _NS

_CB = _C // 8
_STRIPS_PER_W = _BATCH * _CB // _NW
_L = 16
_G = _NPP // _L

_FLAT_PATCH_IDX = np.array([469, 751, 513, 685, 281, 151, 267, 881, 174, 329, 414, 525, 401, 321, 295, 446, 411, 360, 1016, 137, 548, 245, 865, 966, 171, 273, 440, 476, 572, 206, 701, 125, 617, 724, 436, 929, 494, 40, 723, 493, 595, 398, 524, 683, 28, 877, 647, 728, 104, 441, 547, 58, 319, 483, 822, 950, 597, 482, 94, 61, 861, 423, 165, 294, 318, 552, 277, 131, 740, 408, 598, 843, 732, 532, 206, 514, 447, 64, 608, 287, 728, 737, 693, 78, 610, 609, 103, 102, 59, 124, 443, 471, 490, 642, 416, 998, 687, 184, 351, 323, 283, 143, 624, 103, 217, 798, 535, 556, 3, 860, 866, 388, 968, 613, 929, 263, 584, 140, 302, 197, 749, 383, 305, 485, 665, 1019, 218, 893, 229, 114, 67, 842, 69, 548, 540, 764, 969, 969, 695, 341, 232, 991, 65, 111, 629, 207, 860, 990, 792, 594, 759, 865, 958, 179, 12, 119, 917, 109, 710, 617, 953, 777, 746, 432, 755, 519, 885, 851, 266, 96, 188, 513, 577, 631, 415, 586, 95, 595, 810, 208, 508, 608, 706, 422, 456, 229, 518, 144, 38, 665, 87, 975, 657, 759, 82, 791, 976, 788, 279, 192, 903, 395, 765, 13, 659, 236, 805, 494, 789, 633, 568, 542, 887, 503, 891, 610, 264, 812, 273, 293, 893, 978, 137, 478, 806, 116, 529, 657, 646, 442, 122, 678, 775, 859, 193, 855, 868, 562, 737, 1018, 884, 348, 894, 302, 1010, 825, 474, 641, 801, 549, 400, 1000, 667, 89, 799, 220, 304, 299, 292, 472, 709, 387, 396, 815, 650, 882, 826, 1010, 766, 557, 690, 27, 683, 638, 4, 35, 689, 534, 488, 922, 398, 399, 107, 440, 238, 1004, 9, 660, 778, 1003, 359, 827, 310, 409, 159, 264, 478, 320, 760, 444, 462, 1011, 254, 630, 681, 23, 774, 363, 593, 266, 874, 895, 146, 589, 588, 715, 754, 330, 286, 871, 480, 829, 613, 672, 942, 317, 2, 129, 164, 4, 570, 196, 463, 652, 753, 191, 230, 150, 108, 862, 275, 960, 811, 853, 177, 591, 236, 138, 616, 426, 940, 624, 662, 177, 353, 529, 54, 328, 950, 164, 371, 903, 73, 84, 965, 410, 81, 168, 905, 569, 663, 130, 961, 904, 40, 826, 620, 940, 428, 644, 1022, 1014, 228, 785, 5, 337, 1005, 686, 410, 643, 423, 217, 869, 424, 362, 668, 558, 451, 45, 956, 704, 1017, 918, 880, 285, 280, 309, 160, 280, 883, 330, 563, 729, 694, 18, 492, 523, 858, 504, 876, 437, 526, 736, 582, 688, 891, 730, 970, 843, 840, 205, 135, 718, 487, 521, 5, 849, 24, 639, 693, 450, 745, 336, 539, 840, 405, 575, 136, 342, 93, 106, 272, 984, 839, 492, 670, 633, 972, 71, 288, 914, 296, 255, 646, 167, 161, 507, 285, 110, 915, 1013, 632, 405, 367, 417, 51, 1014, 800, 372, 45, 919, 474, 696, 844, 70, 607, 101, 1022, 707, 508, 745, 780, 26, 68, 14, 214, 519, 9, 821, 427, 84, 267, 435, 258, 123, 559, 882, 216, 421, 94, 888, 307, 370, 664, 454, 353, 722, 364, 322, 450, 111, 900, 311, 54, 780, 620, 549, 12, 253, 22, 752, 783, 397, 890, 604, 175, 395, 705, 248, 703, 543, 976, 92, 106, 282, 627, 39, 629, 626, 504, 375, 965, 880, 713, 276, 543, 458, 945, 848, 97, 100, 313, 345, 375, 927, 786, 224, 48, 812, 818, 226, 793, 183, 376, 459, 176, 35, 647, 757, 902, 901, 332, 625, 977, 98, 600, 594, 312, 857, 199, 387, 71, 682, 326, 324, 958, 247, 404, 390, 776, 388, 658, 52, 867, 424, 459, 795, 67, 419, 300, 294, 355, 908, 458, 461, 66, 612, 1020, 668, 246, 776, 244, 120, 531, 76, 794, 764, 789, 404, 808, 160, 460, 931, 157, 442, 65, 29, 869, 944, 1009, 468, 169, 772, 147, 714, 449, 751, 565, 420, 784, 939, 575, 551, 704, 648, 535, 72, 577, 936, 343, 191, 805, 727, 242, 830, 230, 406, 431, 870, 182, 995, 628, 252, 339, 298, 153, 352, 681, 946, 252, 113, 884, 623, 973, 198, 91, 586, 498, 808, 986, 838, 25, 389, 342, 636, 989, 33, 389, 833, 698, 839, 897, 583, 731, 858, 592, 853, 311, 855, 571, 534, 912, 213, 974, 879, 248, 308, 790, 603, 484, 158, 292, 153, 112, 496, 455, 482, 243, 272, 666, 118, 887, 555, 947, 53, 233, 559, 507, 8, 490, 495, 720, 325, 203, 713, 79, 520, 690, 642, 782, 530, 36, 470, 952, 97, 714, 214, 933, 427, 580, 11, 1002, 831, 417, 872, 579, 349, 21, 79, 210, 1001, 615, 877, 362, 169, 517, 418, 232, 133, 390, 219, 588, 566, 909, 105, 238, 803, 443, 173, 56, 109, 926, 1015, 730, 315, 928, 791, 108, 578, 301, 344, 118, 23, 999, 587, 701, 357, 953, 466, 438, 878, 469, 444, 204, 841, 421, 787, 295, 788, 10, 373, 828, 135, 171, 988, 692, 876, 816, 246, 799, 889, 604, 88, 370, 859, 654, 368, 454, 844, 69, 896, 846, 208, 801, 211, 634, 814, 707, 78, 921, 460, 946, 31, 371, 734, 166, 692, 530, 212, 319, 582, 939, 818, 852, 658, 163, 1, 183, 197, 344, 684, 963, 738, 854, 216, 467, 602, 803, 57, 688, 671, 250, 813, 425, 926, 406, 185, 522, 785, 31, 215, 152, 143, 566, 983, 568, 194, 76, 600, 73, 7, 995, 502, 30, 1003, 456, 916, 721, 816, 123, 86, 158, 355, 523, 807, 663, 743, 8, 131, 209, 234, 797, 75, 39, 994, 453, 385, 545, 711, 612, 666, 274, 895, 479, 479, 618, 121, 659, 377, 419, 56, 373, 265, 994, 312, 139, 957, 457, 697, 418, 1021, 497, 638, 971, 510, 188, 392, 462, 126, 41, 1017, 773, 673, 133, 303, 289, 425, 59, 299, 391, 998, 226, 705, 369, 134, 931, 567, 1, 828, 159, 429, 927, 771, 325, 403, 412, 515, 807, 676, 528, 341, 546, 13, 898, 125, 117, 851, 361, 321, 15, 471, 172, 220, 366, 240, 1021, 565, 413, 562, 753, 1020, 547, 259, 17, 132, 437, 991, 439, 249, 368, 794, 468, 413, 357], dtype=np.int32)

_COL_IDX = _FLAT_PATCH_IDX.reshape(_NPP, _N_PATHS).T.reshape(-1).copy()


def _lane_gather_body(x_hbm, idx_hbm, out_hbm, idx_v, in_v, out_v,
                      in_sems, out_sems):
    wid = lax.axis_index("s") * _NC + lax.axis_index("c")

    pltpu.sync_copy(idx_hbm, idx_v)

    def fetch(slot, k):
        return pltpu.async_copy(x_hbm.at[pl.ds(k * 8, 16), :],
                                in_v.at[slot], in_sems.at[slot])

    def flush(slot, k):
        b = k // _CB
        cb = k - b * _CB
        for n in range(_N_PATHS):
            ko = (b * _N_PATHS + n) * _CB + cb
            pltpu.async_copy(out_v.at[slot, n],
                             out_hbm.at[pl.ds(ko * 8, 16), :],
                             out_sems.at[slot, n])

    def compute(slot):
        for s2 in range(2):
            @plsc.parallel_loop(0, _N_PATHS * _G, unroll=8)
            def _(gi):
                colv = idx_v[pl.ds(gi * _L, _L)]
                n = lax.shift_right_logical(gi, 5)
                joff = lax.bitwise_and(gi, 31) * _L
                for c in range(8):
                    rowv = jnp.full((_L,), 8 * s2 + c, jnp.int32)
                    val = plsc.load_gather(in_v.at[slot], [rowv, colv])
                    out_v[slot, n, 8 * s2 + c, pl.ds(joff, _L)] = val

    base = wid * _STRIPS_PER_W
    npairs = _STRIPS_PER_W // 2

    fetch(0, base + 0)
    fetch(1, base + 2)

    def pair_step(q, carry):
        slot = lax.rem(q, 2)
        k = base + 2 * q
        pltpu.make_async_copy(x_hbm.at[pl.ds(k * 8, 16), :],
                              in_v.at[slot], in_sems.at[slot]).wait()
        @pl.when(q >= 2)
        def _():
            for n in range(_N_PATHS):
                pltpu.make_async_copy(out_v.at[slot, n],
                                      out_hbm.at[pl.ds(0, 16), :],
                                      out_sems.at[slot, n]).wait()
        compute(slot)
        flush(slot, k)
        @pl.when(q + 2 < npairs)
        def _():
            fetch(slot, k + 4)
        return carry

    lax.fori_loop(0, npairs, pair_step, 0)

    for q in range(npairs - 2, npairs):
        slot = q % 2
        for n in range(_N_PATHS):
            pltpu.make_async_copy(out_v.at[slot, n],
                                  out_hbm.at[pl.ds(0, 16), :],
                                  out_sems.at[slot, n]).wait()


_lane_gather = pl.kernel(
    _lane_gather_body,
    out_type=jax.ShapeDtypeStruct((_BATCH * _N_PATHS * _C, _NPP), jnp.float32),
    mesh=plsc.VectorSubcoreMesh(core_axis_name="c", subcore_axis_name="s"),
    compiler_params=pltpu.CompilerParams(use_tc_tiling_on_sc=True,
                                         needs_layout_passes=False),
    scratch_types=[
        pltpu.VMEM((_N_PATHS * _NPP,), jnp.int32),
        pltpu.VMEM((2, 16, _NUM_PATCHES), jnp.float32),
        pltpu.VMEM((2, _N_PATHS, 16, _NPP), jnp.float32),
        pltpu.SemaphoreType.DMA((2,)),
        pltpu.SemaphoreType.DMA((2, _N_PATHS)),
    ],
)


def kernel(inputs):
    x_t = jnp.transpose(inputs, (0, 2, 1)).reshape(_BATCH * _C, _NUM_PATCHES)
    col_idx = jnp.asarray(_COL_IDX)
    out_t = _lane_gather(x_t, col_idx)
    out_t = out_t.reshape(_BATCH, _N_PATHS, _C, _NPP)
    return jnp.transpose(out_t, (0, 3, 1, 2))


# speedup vs baseline: 1.1706x; 1.0086x over previous
import jax
import jax.numpy as jnp
import numpy as np
from jax import lax
from jax.experimental import pallas as pl
from jax.experimental.pallas import tpu as pltpu
from jax.experimental.pallas import tpu_sc as plsc

_NUM_PATCHES = 1024
_N_PATHS = 2
_NPP = 512
_BATCH = 64
_C = 192

_NC = 2
_NS = 16
_NW = _NC * _NS

_CB = _C // 8
_STRIPS_PER_W = _BATCH * _CB // _NW
_L = 16
_G = _NPP // _L

_FLAT_PATCH_IDX = np.array([469, 751, 513, 685, 281, 151, 267, 881, 174, 329, 414, 525, 401, 321, 295, 446, 411, 360, 1016, 137, 548, 245, 865, 966, 171, 273, 440, 476, 572, 206, 701, 125, 617, 724, 436, 929, 494, 40, 723, 493, 595, 398, 524, 683, 28, 877, 647, 728, 104, 441, 547, 58, 319, 483, 822, 950, 597, 482, 94, 61, 861, 423, 165, 294, 318, 552, 277, 131, 740, 408, 598, 843, 732, 532, 206, 514, 447, 64, 608, 287, 728, 737, 693, 78, 610, 609, 103, 102, 59, 124, 443, 471, 490, 642, 416, 998, 687, 184, 351, 323, 283, 143, 624, 103, 217, 798, 535, 556, 3, 860, 866, 388, 968, 613, 929, 263, 584, 140, 302, 197, 749, 383, 305, 485, 665, 1019, 218, 893, 229, 114, 67, 842, 69, 548, 540, 764, 969, 969, 695, 341, 232, 991, 65, 111, 629, 207, 860, 990, 792, 594, 759, 865, 958, 179, 12, 119, 917, 109, 710, 617, 953, 777, 746, 432, 755, 519, 885, 851, 266, 96, 188, 513, 577, 631, 415, 586, 95, 595, 810, 208, 508, 608, 706, 422, 456, 229, 518, 144, 38, 665, 87, 975, 657, 759, 82, 791, 976, 788, 279, 192, 903, 395, 765, 13, 659, 236, 805, 494, 789, 633, 568, 542, 887, 503, 891, 610, 264, 812, 273, 293, 893, 978, 137, 478, 806, 116, 529, 657, 646, 442, 122, 678, 775, 859, 193, 855, 868, 562, 737, 1018, 884, 348, 894, 302, 1010, 825, 474, 641, 801, 549, 400, 1000, 667, 89, 799, 220, 304, 299, 292, 472, 709, 387, 396, 815, 650, 882, 826, 1010, 766, 557, 690, 27, 683, 638, 4, 35, 689, 534, 488, 922, 398, 399, 107, 440, 238, 1004, 9, 660, 778, 1003, 359, 827, 310, 409, 159, 264, 478, 320, 760, 444, 462, 1011, 254, 630, 681, 23, 774, 363, 593, 266, 874, 895, 146, 589, 588, 715, 754, 330, 286, 871, 480, 829, 613, 672, 942, 317, 2, 129, 164, 4, 570, 196, 463, 652, 753, 191, 230, 150, 108, 862, 275, 960, 811, 853, 177, 591, 236, 138, 616, 426, 940, 624, 662, 177, 353, 529, 54, 328, 950, 164, 371, 903, 73, 84, 965, 410, 81, 168, 905, 569, 663, 130, 961, 904, 40, 826, 620, 940, 428, 644, 1022, 1014, 228, 785, 5, 337, 1005, 686, 410, 643, 423, 217, 869, 424, 362, 668, 558, 451, 45, 956, 704, 1017, 918, 880, 285, 280, 309, 160, 280, 883, 330, 563, 729, 694, 18, 492, 523, 858, 504, 876, 437, 526, 736, 582, 688, 891, 730, 970, 843, 840, 205, 135, 718, 487, 521, 5, 849, 24, 639, 693, 450, 745, 336, 539, 840, 405, 575, 136, 342, 93, 106, 272, 984, 839, 492, 670, 633, 972, 71, 288, 914, 296, 255, 646, 167, 161, 507, 285, 110, 915, 1013, 632, 405, 367, 417, 51, 1014, 800, 372, 45, 919, 474, 696, 844, 70, 607, 101, 1022, 707, 508, 745, 780, 26, 68, 14, 214, 519, 9, 821, 427, 84, 267, 435, 258, 123, 559, 882, 216, 421, 94, 888, 307, 370, 664, 454, 353, 722, 364, 322, 450, 111, 900, 311, 54, 780, 620, 549, 12, 253, 22, 752, 783, 397, 890, 604, 175, 395, 705, 248, 703, 543, 976, 92, 106, 282, 627, 39, 629, 626, 504, 375, 965, 880, 713, 276, 543, 458, 945, 848, 97, 100, 313, 345, 375, 927, 786, 224, 48, 812, 818, 226, 793, 183, 376, 459, 176, 35, 647, 757, 902, 901, 332, 625, 977, 98, 600, 594, 312, 857, 199, 387, 71, 682, 326, 324, 958, 247, 404, 390, 776, 388, 658, 52, 867, 424, 459, 795, 67, 419, 300, 294, 355, 908, 458, 461, 66, 612, 1020, 668, 246, 776, 244, 120, 531, 76, 794, 764, 789, 404, 808, 160, 460, 931, 157, 442, 65, 29, 869, 944, 1009, 468, 169, 772, 147, 714, 449, 751, 565, 420, 784, 939, 575, 551, 704, 648, 535, 72, 577, 936, 343, 191, 805, 727, 242, 830, 230, 406, 431, 870, 182, 995, 628, 252, 339, 298, 153, 352, 681, 946, 252, 113, 884, 623, 973, 198, 91, 586, 498, 808, 986, 838, 25, 389, 342, 636, 989, 33, 389, 833, 698, 839, 897, 583, 731, 858, 592, 853, 311, 855, 571, 534, 912, 213, 974, 879, 248, 308, 790, 603, 484, 158, 292, 153, 112, 496, 455, 482, 243, 272, 666, 118, 887, 555, 947, 53, 233, 559, 507, 8, 490, 495, 720, 325, 203, 713, 79, 520, 690, 642, 782, 530, 36, 470, 952, 97, 714, 214, 933, 427, 580, 11, 1002, 831, 417, 872, 579, 349, 21, 79, 210, 1001, 615, 877, 362, 169, 517, 418, 232, 133, 390, 219, 588, 566, 909, 105, 238, 803, 443, 173, 56, 109, 926, 1015, 730, 315, 928, 791, 108, 578, 301, 344, 118, 23, 999, 587, 701, 357, 953, 466, 438, 878, 469, 444, 204, 841, 421, 787, 295, 788, 10, 373, 828, 135, 171, 988, 692, 876, 816, 246, 799, 889, 604, 88, 370, 859, 654, 368, 454, 844, 69, 896, 846, 208, 801, 211, 634, 814, 707, 78, 921, 460, 946, 31, 371, 734, 166, 692, 530, 212, 319, 582, 939, 818, 852, 658, 163, 1, 183, 197, 344, 684, 963, 738, 854, 216, 467, 602, 803, 57, 688, 671, 250, 813, 425, 926, 406, 185, 522, 785, 31, 215, 152, 143, 566, 983, 568, 194, 76, 600, 73, 7, 995, 502, 30, 1003, 456, 916, 721, 816, 123, 86, 158, 355, 523, 807, 663, 743, 8, 131, 209, 234, 797, 75, 39, 994, 453, 385, 545, 711, 612, 666, 274, 895, 479, 479, 618, 121, 659, 377, 419, 56, 373, 265, 994, 312, 139, 957, 457, 697, 418, 1021, 497, 638, 971, 510, 188, 392, 462, 126, 41, 1017, 773, 673, 133, 303, 289, 425, 59, 299, 391, 998, 226, 705, 369, 134, 931, 567, 1, 828, 159, 429, 927, 771, 325, 403, 412, 515, 807, 676, 528, 341, 546, 13, 898, 125, 117, 851, 361, 321, 15, 471, 172, 220, 366, 240, 1021, 565, 413, 562, 753, 1020, 547, 259, 17, 132, 437, 991, 439, 249, 368, 794, 468, 413, 357], dtype=np.int32)

_COL_IDX = _FLAT_PATCH_IDX.reshape(_NPP, _N_PATHS).T.reshape(-1).copy()


def _lane_gather_body(x_hbm, idx_hbm, out_hbm, idx_v, in_v, out_v,
                      in_sems, out_sems):
    wid = lax.axis_index("s") * _NC + lax.axis_index("c")

    pltpu.sync_copy(idx_hbm, idx_v)

    def fetch(slot, k):
        return pltpu.async_copy(x_hbm.at[pl.ds(k * 8, 24), :],
                                in_v.at[slot], in_sems.at[slot])

    def flush(slot, k):
        b = k // _CB
        cb = k - b * _CB
        for n in range(_N_PATHS):
            ko = (b * _N_PATHS + n) * _CB + cb
            pltpu.async_copy(out_v.at[slot, n],
                             out_hbm.at[pl.ds(ko * 8, 24), :],
                             out_sems.at[slot, n])

    def compute(slot):
        for s2 in range(3):
            @plsc.parallel_loop(0, _N_PATHS * _G, unroll=8)
            def _(gi):
                colv = idx_v[pl.ds(gi * _L, _L)]
                n = lax.shift_right_logical(gi, 5)
                joff = lax.bitwise_and(gi, 31) * _L
                for c in range(8):
                    rowv = jnp.full((_L,), 8 * s2 + c, jnp.int32)
                    val = plsc.load_gather(in_v.at[slot], [rowv, colv])
                    out_v[slot, n, 8 * s2 + c, pl.ds(joff, _L)] = val

    base = wid * _STRIPS_PER_W
    npairs = _STRIPS_PER_W // 3

    fetch(0, base + 0)
    fetch(1, base + 3)

    def pair_step(q, carry):
        slot = lax.rem(q, 2)
        k = base + 3 * q
        pltpu.make_async_copy(x_hbm.at[pl.ds(k * 8, 24), :],
                              in_v.at[slot], in_sems.at[slot]).wait()
        @pl.when(q >= 2)
        def _():
            for n in range(_N_PATHS):
                pltpu.make_async_copy(out_v.at[slot, n],
                                      out_hbm.at[pl.ds(0, 24), :],
                                      out_sems.at[slot, n]).wait()
        compute(slot)
        flush(slot, k)
        @pl.when(q + 2 < npairs)
        def _():
            fetch(slot, k + 6)
        return carry

    lax.fori_loop(0, npairs, pair_step, 0)

    for q in range(npairs - 2, npairs):
        slot = q % 2
        for n in range(_N_PATHS):
            pltpu.make_async_copy(out_v.at[slot, n],
                                  out_hbm.at[pl.ds(0, 24), :],
                                  out_sems.at[slot, n]).wait()


_lane_gather = pl.kernel(
    _lane_gather_body,
    out_type=jax.ShapeDtypeStruct((_BATCH * _N_PATHS * _C, _NPP), jnp.float32),
    mesh=plsc.VectorSubcoreMesh(core_axis_name="c", subcore_axis_name="s"),
    compiler_params=pltpu.CompilerParams(use_tc_tiling_on_sc=True,
                                         needs_layout_passes=False),
    scratch_types=[
        pltpu.VMEM((_N_PATHS * _NPP,), jnp.int32),
        pltpu.VMEM((2, 24, _NUM_PATCHES), jnp.float32),
        pltpu.VMEM((2, _N_PATHS, 24, _NPP), jnp.float32),
        pltpu.SemaphoreType.DMA((2,)),
        pltpu.SemaphoreType.DMA((2, _N_PATHS)),
    ],
)


def kernel(inputs):
    x_t = jnp.transpose(inputs, (0, 2, 1)).reshape(_BATCH * _C, _NUM_PATCHES)
    col_idx = jnp.asarray(_COL_IDX)
    out_t = _lane_gather(x_t, col_idx)
    out_t = out_t.reshape(_BATCH, _N_PATHS, _C, _NPP)
    return jnp.transpose(out_t, (0, 3, 1, 2))
